# baseline (speedup 1.0000x reference)
import jax
import jax.numpy as jnp
from jax.experimental import pallas as pl
from jax.experimental.pallas import tpu as pltpu
from jax.experimental.shard_map import shard_map
from jax.sharding import PartitionSpec as P


def _round_up(n, m):
    return ((n + m - 1) // m) * m


def _lstm_kernel(x_ref,
                 wih_ref,
                 whh_ref,
                 b_ref,
                 w1_ref,
                 b1_ref,
                 w2_ref,
                 b2_ref,
                 out_ref,
                 h2_s,
                 c_s):
    BB, TC, D = x_ref.shape
    H = whh_ref.shape[0]
    j = pl.program_id(1)
    NT = pl.num_programs(1)

    wih = wih_ref[...]
    whh = whh_ref[...]
    bias = b_ref[...]

    xt = jnp.transpose(x_ref[...], (1, 0, 2)).astype(jnp.bfloat16)

    @pl.when(j == 0)
    def _init():
        h2_s[...] = jnp.zeros_like(h2_s)
        c_s[...] = jnp.zeros_like(c_s)

    def step(gates, c):
        th_ig = jnp.tanh(gates[:, :2 * H])
        m = th_ig[:, :H] * th_ig[:, H:] + th_ig[:, H:]
        th_f = jnp.tanh(gates[:, 2 * H:3 * H])
        c = 0.5 * (th_f * c + c + m)
        th_o = jnp.tanh(gates[:, 3 * H:])
        h2 = jnp.tanh(c) * (th_o + 1.0)
        return h2, c

    h2_bf = h2_s[...]
    c = c_s[...]
    h2 = None
    for k in range(TC):
        gates = (jnp.dot(xt[k], wih, preferred_element_type=jnp.float32)
                 + jnp.dot(h2_bf, whh, preferred_element_type=jnp.float32)
                 + bias)
        h2, c = step(gates, c)
        h2_bf = h2.astype(jnp.bfloat16)
    h2_s[...] = h2_bf
    c_s[...] = c

    @pl.when(j == NT - 1)
    def _head():
        z1 = (jnp.dot(h2, w1_ref[...], preferred_element_type=jnp.float32)
              + b1_ref[...])
        z1 = jnp.maximum(z1, 0.0)
        z2 = (jnp.dot(z1, w2_ref[...], preferred_element_type=jnp.float32)
              + b2_ref[...])
        out_ref[...] = jax.nn.sigmoid(z2)


def _forward(x, wih_s, whh_s, b_s, w1_s, b1, w2p, b2p):
    B, T, D = x.shape
    H = whh_s.shape[0]
    G = 4 * H
    F1 = w1_s.shape[1]
    OP = w2p.shape[1]

    batch_block = min(512, _round_up(B, 8))
    batch_block = max(8, _round_up(batch_block, 8))
    B_pad = _round_up(B, batch_block)
    if B_pad != B:
        x = jnp.pad(x, ((0, B_pad - B), (0, 0), (0, 0)))

    TC = 16
    while T % TC:
        TC -= 1
    NT = T // TC

    nb = B_pad // batch_block

    out = pl.pallas_call(
        _lstm_kernel,
        out_shape=jax.ShapeDtypeStruct((B_pad, OP), jnp.float32),
        grid_spec=pltpu.PrefetchScalarGridSpec(
            num_scalar_prefetch=0,
            grid=(nb, NT),
            in_specs=[
                pl.BlockSpec((batch_block, TC, D), lambda i, j: (i, j, 0)),
                pl.BlockSpec((D, G), lambda i, j: (0, 0)),
                pl.BlockSpec((H, G), lambda i, j: (0, 0)),
                pl.BlockSpec((1, G), lambda i, j: (0, 0)),
                pl.BlockSpec((H, F1), lambda i, j: (0, 0)),
                pl.BlockSpec((1, F1), lambda i, j: (0, 0)),
                pl.BlockSpec((F1, OP), lambda i, j: (0, 0)),
                pl.BlockSpec((1, OP), lambda i, j: (0, 0)),
            ],
            out_specs=pl.BlockSpec((batch_block, OP), lambda i, j: (i, 0)),
            scratch_shapes=[pltpu.VMEM((batch_block, H), jnp.bfloat16),
                            pltpu.VMEM((batch_block, H), jnp.float32)],
        ),
        compiler_params=pltpu.CompilerParams(
            dimension_semantics=("parallel", "arbitrary"),
            vmem_limit_bytes=100 * 1024 * 1024,
        ),
    )(x, wih_s, whh_s, b_s, w1_s, b1, w2p, b2p)

    return out[:B]


def kernel(x, wih_t, whh_t, b_lstm, w1_t, b1, w2_t, b2):
    B, T, D = x.shape
    H = whh_t.shape[0]
    G = 4 * H
    O = w2_t.shape[1]
    OP = _round_up(O, 128)

    lane = jnp.arange(G)
    g_lane = (lane >= 2 * H) & (lane < 3 * H)
    colscale = jnp.where(g_lane, 1.0, 0.5).astype(jnp.float32)

    def reorder(w):
        return jnp.concatenate(
            [w[:, :H], w[:, 2 * H:3 * H], w[:, H:2 * H], w[:, 3 * H:]], axis=1)

    wih_s = reorder(wih_t * colscale[None, :]).astype(jnp.bfloat16)
    whh_s = reorder(0.5 * whh_t * colscale[None, :]).astype(jnp.bfloat16)
    b_s = reorder(b_lstm * colscale[None, :])
    w1_s = 0.5 * w1_t
    w2p = jnp.pad(w2_t, ((0, 0), (0, OP - O)))
    b2p = jnp.pad(b2, ((0, 0), (0, OP - O)))

    n_dev = len(jax.devices())
    n_shards = 2 if (n_dev >= 2 and B % 2 == 0) else 1
    if n_shards > 1:
        mesh = jax.make_mesh((n_shards,), ("b",),
                             axis_types=(jax.sharding.AxisType.Auto,))
        fwd = shard_map(
            _forward, mesh=mesh,
            in_specs=(P("b"), P(), P(), P(), P(), P(), P(), P()),
            out_specs=P("b"), check_rep=False)
    else:
        fwd = _forward
    out = fwd(x, wih_s, whh_s, b_s, w1_s, b1, w2p, b2p)
    return out[:, :O]


# speedup vs baseline: 4.8035x; 4.8035x over previous
import jax
import jax.numpy as jnp
from jax.experimental import pallas as pl
from jax.experimental.pallas import tpu as pltpu


def _round_up(n, m):
    return ((n + m - 1) // m) * m


def _lstm_kernel(x_ref,
                 wih_ref,
                 whh_ref,
                 b_ref,
                 w1_ref,
                 b1_ref,
                 w2_ref,
                 b2_ref,
                 out_ref,
                 h2_s,
                 c_s):
    BB, TC, D = x_ref.shape
    H = whh_ref.shape[0]
    j = pl.program_id(1)
    NT = pl.num_programs(1)

    wih = wih_ref[...]
    whh = whh_ref[...]
    bias = b_ref[...]

    xt = jnp.transpose(x_ref[...], (1, 0, 2)).astype(jnp.bfloat16)

    @pl.when(j == 0)
    def _init():
        h2_s[...] = jnp.zeros_like(h2_s)
        c_s[...] = jnp.zeros_like(c_s)

    def step(gates, c):
        th_ig = jnp.tanh(gates[:, :2 * H])
        m = th_ig[:, :H] * th_ig[:, H:] + th_ig[:, H:]
        th_f = jnp.tanh(gates[:, 2 * H:3 * H])
        c = 0.5 * (th_f * c + c + m)
        th_o = jnp.tanh(gates[:, 3 * H:])
        h2 = jnp.tanh(c) * (th_o + 1.0)
        return h2, c

    h2_bf = h2_s[...]
    c = c_s[...]
    h2 = None
    for k in range(TC):
        gates = (jnp.dot(xt[k], wih, preferred_element_type=jnp.float32)
                 + jnp.dot(h2_bf, whh, preferred_element_type=jnp.float32)
                 + bias)
        h2, c = step(gates, c)
        h2_bf = h2.astype(jnp.bfloat16)
    h2_s[...] = h2_bf
    c_s[...] = c

    @pl.when(j == NT - 1)
    def _head():
        z1 = (jnp.dot(h2, w1_ref[...], preferred_element_type=jnp.float32)
              + b1_ref[...])
        z1 = jnp.maximum(z1, 0.0)
        z2 = (jnp.dot(z1, w2_ref[...], preferred_element_type=jnp.float32)
              + b2_ref[...])
        out_ref[...] = jax.nn.sigmoid(z2)


def kernel(x, wih_t, whh_t, b_lstm, w1_t, b1, w2_t, b2):
    B, T, D = x.shape
    H = whh_t.shape[0]
    G = 4 * H
    F1 = w1_t.shape[1]
    O = w2_t.shape[1]

    batch_block = min(1024, _round_up(B, 8))
    batch_block = max(8, _round_up(batch_block, 8))
    B_pad = _round_up(B, batch_block)
    OP = _round_up(O, 128)
    if B_pad != B:
        x = jnp.pad(x, ((0, B_pad - B), (0, 0), (0, 0)))
    w2p = jnp.pad(w2_t, ((0, 0), (0, OP - O)))
    b2p = jnp.pad(b2, ((0, 0), (0, OP - O)))

    TC = 16
    while T % TC:
        TC -= 1
    NT = T // TC

    lane = jnp.arange(G)
    g_lane = (lane >= 2 * H) & (lane < 3 * H)
    colscale = jnp.where(g_lane, 1.0, 0.5).astype(jnp.float32)

    def reorder(w):
        return jnp.concatenate(
            [w[:, :H], w[:, 2 * H:3 * H], w[:, H:2 * H], w[:, 3 * H:]], axis=1)

    wih_s = reorder(wih_t * colscale[None, :]).astype(jnp.bfloat16)
    whh_s = reorder(0.5 * whh_t * colscale[None, :]).astype(jnp.bfloat16)
    b_s = reorder(b_lstm * colscale[None, :])
    w1_s = 0.5 * w1_t

    nb = B_pad // batch_block

    out = pl.pallas_call(
        _lstm_kernel,
        out_shape=jax.ShapeDtypeStruct((B_pad, OP), jnp.float32),
        grid_spec=pltpu.PrefetchScalarGridSpec(
            num_scalar_prefetch=0,
            grid=(nb, NT),
            in_specs=[
                pl.BlockSpec((batch_block, TC, D), lambda i, j: (i, j, 0)),
                pl.BlockSpec((D, G), lambda i, j: (0, 0)),
                pl.BlockSpec((H, G), lambda i, j: (0, 0)),
                pl.BlockSpec((1, G), lambda i, j: (0, 0)),
                pl.BlockSpec((H, F1), lambda i, j: (0, 0)),
                pl.BlockSpec((1, F1), lambda i, j: (0, 0)),
                pl.BlockSpec((F1, OP), lambda i, j: (0, 0)),
                pl.BlockSpec((1, OP), lambda i, j: (0, 0)),
            ],
            out_specs=pl.BlockSpec((batch_block, OP), lambda i, j: (i, 0)),
            scratch_shapes=[pltpu.VMEM((batch_block, H), jnp.bfloat16),
                            pltpu.VMEM((batch_block, H), jnp.float32)],
        ),
        compiler_params=pltpu.CompilerParams(
            dimension_semantics=("parallel", "arbitrary"),
            vmem_limit_bytes=100 * 1024 * 1024,
        ),
    )(x, wih_s, whh_s, b_s, w1_s, b1, w2p, b2p)

    return out[:B, :O]


# speedup vs baseline: 4.9846x; 1.0377x over previous
import jax
import jax.numpy as jnp
from jax.experimental import pallas as pl
from jax.experimental.pallas import tpu as pltpu


def _round_up(n, m):
    return ((n + m - 1) // m) * m


def _lstm_kernel(x_ref,
                 wih_ref,
                 whh_ref,
                 b_ref,
                 w1_ref,
                 b1_ref,
                 w2_ref,
                 b2_ref,
                 out_ref,
                 h2_s,
                 c_s):
    BB, TC, D = x_ref.shape
    H = whh_ref.shape[0]
    j = pl.program_id(1)
    NT = pl.num_programs(1)

    wih = wih_ref[...]
    whh = whh_ref[...]
    bias = b_ref[...]

    xt = jnp.transpose(x_ref[...], (1, 0, 2)).astype(jnp.bfloat16)

    @pl.when(j == 0)
    def _init():
        h2_s[...] = jnp.zeros_like(h2_s)
        c_s[...] = jnp.zeros_like(c_s)

    def step(gates, c):
        th_ig = jnp.tanh(gates[:, :2 * H])
        m = th_ig[:, :H] * th_ig[:, H:] + th_ig[:, H:]
        th_f = jnp.tanh(gates[:, 2 * H:3 * H])
        c = 0.5 * (th_f * c + c + m)
        th_o = jnp.tanh(gates[:, 3 * H:])
        h2 = jnp.tanh(c) * (th_o + 1.0)
        return h2, c

    h2_bf = h2_s[...]
    c = c_s[...]
    h2 = None
    for k in range(TC):
        gates = (jnp.dot(xt[k], wih, preferred_element_type=jnp.float32)
                 + jnp.dot(h2_bf, whh, preferred_element_type=jnp.float32)
                 + bias)
        h2, c = step(gates, c)
        h2_bf = h2.astype(jnp.bfloat16)
    h2_s[...] = h2_bf
    c_s[...] = c

    @pl.when(j == NT - 1)
    def _head():
        z1 = (jnp.dot(h2, w1_ref[...], preferred_element_type=jnp.float32)
              + b1_ref[...])
        z1 = jnp.maximum(z1, 0.0)
        z2 = (jnp.dot(z1, w2_ref[...], preferred_element_type=jnp.float32)
              + b2_ref[...])
        out_ref[...] = jax.nn.sigmoid(z2)


def kernel(x, wih_t, whh_t, b_lstm, w1_t, b1, w2_t, b2):
    B, T, D = x.shape
    H = whh_t.shape[0]
    G = 4 * H
    F1 = w1_t.shape[1]
    O = w2_t.shape[1]

    batch_block = min(512, _round_up(B, 8))
    batch_block = max(8, _round_up(batch_block, 8))
    B_pad = _round_up(B, batch_block)
    OP = _round_up(O, 128)
    if B_pad != B:
        x = jnp.pad(x, ((0, B_pad - B), (0, 0), (0, 0)))
    w2p = jnp.pad(w2_t, ((0, 0), (0, OP - O)))
    b2p = jnp.pad(b2, ((0, 0), (0, OP - O)))

    TC = 16
    while T % TC:
        TC -= 1
    NT = T // TC

    lane = jnp.arange(G)
    g_lane = (lane >= 2 * H) & (lane < 3 * H)
    colscale = jnp.where(g_lane, 1.0, 0.5).astype(jnp.float32)

    def reorder(w):
        return jnp.concatenate(
            [w[:, :H], w[:, 2 * H:3 * H], w[:, H:2 * H], w[:, 3 * H:]], axis=1)

    wih_s = reorder(wih_t * colscale[None, :]).astype(jnp.bfloat16)
    whh_s = reorder(0.5 * whh_t * colscale[None, :]).astype(jnp.bfloat16)
    b_s = reorder(b_lstm * colscale[None, :])
    w1_s = 0.5 * w1_t

    nb = B_pad // batch_block

    out = pl.pallas_call(
        _lstm_kernel,
        out_shape=jax.ShapeDtypeStruct((B_pad, OP), jnp.float32),
        grid_spec=pltpu.PrefetchScalarGridSpec(
            num_scalar_prefetch=0,
            grid=(nb, NT),
            in_specs=[
                pl.BlockSpec((batch_block, TC, D), lambda i, j: (i, j, 0)),
                pl.BlockSpec((D, G), lambda i, j: (0, 0)),
                pl.BlockSpec((H, G), lambda i, j: (0, 0)),
                pl.BlockSpec((1, G), lambda i, j: (0, 0)),
                pl.BlockSpec((H, F1), lambda i, j: (0, 0)),
                pl.BlockSpec((1, F1), lambda i, j: (0, 0)),
                pl.BlockSpec((F1, OP), lambda i, j: (0, 0)),
                pl.BlockSpec((1, OP), lambda i, j: (0, 0)),
            ],
            out_specs=pl.BlockSpec((batch_block, OP), lambda i, j: (i, 0)),
            scratch_shapes=[pltpu.VMEM((batch_block, H), jnp.bfloat16),
                            pltpu.VMEM((batch_block, H), jnp.float32)],
        ),
        compiler_params=pltpu.CompilerParams(
            dimension_semantics=("parallel", "arbitrary"),
            vmem_limit_bytes=100 * 1024 * 1024,
        ),
    )(x, wih_s, whh_s, b_s, w1_s, b1, w2p, b2p)

    return out[:B, :O]


# speedup vs baseline: 5.3374x; 1.0708x over previous
import jax
import jax.numpy as jnp
from jax.experimental import pallas as pl
from jax.experimental.pallas import tpu as pltpu


def _round_up(n, m):
    return ((n + m - 1) // m) * m


def _lstm_kernel(x_ref,
                 wih_ref,
                 whh_ref,
                 b_ref,
                 w1_ref,
                 b1_ref,
                 w2_ref,
                 b2_ref,
                 out_ref,
                 h2_s,
                 c_s):
    BB, TC, D = x_ref.shape
    H = whh_ref.shape[0]
    j = pl.program_id(1)
    NT = pl.num_programs(1)

    wih = wih_ref[...]
    whh = whh_ref[...]
    bias = b_ref[...]

    xt = jnp.transpose(x_ref[...].astype(jnp.bfloat16), (1, 0, 2))

    @pl.when(j == 0)
    def _init():
        h2_s[...] = jnp.zeros_like(h2_s)
        c_s[...] = jnp.zeros_like(c_s)

    def step(gates, c):
        th_ig = jnp.tanh(gates[:, :2 * H])
        m = th_ig[:, :H] * th_ig[:, H:] + th_ig[:, H:]
        th_f = jnp.tanh(gates[:, 2 * H:3 * H])
        c = 0.5 * (th_f * c + c + m)
        th_o = jnp.tanh(gates[:, 3 * H:])
        h2 = jnp.tanh(c) * (th_o + 1.0)
        return h2, c

    h2_bf = h2_s[...]
    c = c_s[...]
    h2 = None
    for k in range(TC):
        gates = (jnp.dot(xt[k], wih, preferred_element_type=jnp.float32)
                 + jnp.dot(h2_bf, whh, preferred_element_type=jnp.float32)
                 + bias)
        h2, c = step(gates, c)
        h2_bf = h2.astype(jnp.bfloat16)
    h2_s[...] = h2_bf
    c_s[...] = c

    @pl.when(j == NT - 1)
    def _head():
        z1 = (jnp.dot(h2, w1_ref[...], preferred_element_type=jnp.float32)
              + b1_ref[...])
        z1 = jnp.maximum(z1, 0.0)
        z2 = (jnp.dot(z1, w2_ref[...], preferred_element_type=jnp.float32)
              + b2_ref[...])
        out_ref[...] = jax.nn.sigmoid(z2)


def kernel(x, wih_t, whh_t, b_lstm, w1_t, b1, w2_t, b2):
    B, T, D = x.shape
    H = whh_t.shape[0]
    G = 4 * H
    F1 = w1_t.shape[1]
    O = w2_t.shape[1]

    batch_block = min(512, _round_up(B, 8))
    batch_block = max(8, _round_up(batch_block, 8))
    B_pad = _round_up(B, batch_block)
    OP = _round_up(O, 128)
    if B_pad != B:
        x = jnp.pad(x, ((0, B_pad - B), (0, 0), (0, 0)))
    w2p = jnp.pad(w2_t, ((0, 0), (0, OP - O)))
    b2p = jnp.pad(b2, ((0, 0), (0, OP - O)))

    TC = 16
    while T % TC:
        TC -= 1
    NT = T // TC

    lane = jnp.arange(G)
    g_lane = (lane >= 2 * H) & (lane < 3 * H)
    colscale = jnp.where(g_lane, 1.0, 0.5).astype(jnp.float32)

    def reorder(w):
        return jnp.concatenate(
            [w[:, :H], w[:, 2 * H:3 * H], w[:, H:2 * H], w[:, 3 * H:]], axis=1)

    wih_s = reorder(wih_t * colscale[None, :]).astype(jnp.bfloat16)
    whh_s = reorder(0.5 * whh_t * colscale[None, :]).astype(jnp.bfloat16)
    b_s = reorder(b_lstm * colscale[None, :])
    w1_s = 0.5 * w1_t

    nb = B_pad // batch_block

    out = pl.pallas_call(
        _lstm_kernel,
        out_shape=jax.ShapeDtypeStruct((B_pad, OP), jnp.float32),
        grid_spec=pltpu.PrefetchScalarGridSpec(
            num_scalar_prefetch=0,
            grid=(nb, NT),
            in_specs=[
                pl.BlockSpec((batch_block, TC, D), lambda i, j: (i, j, 0)),
                pl.BlockSpec((D, G), lambda i, j: (0, 0)),
                pl.BlockSpec((H, G), lambda i, j: (0, 0)),
                pl.BlockSpec((1, G), lambda i, j: (0, 0)),
                pl.BlockSpec((H, F1), lambda i, j: (0, 0)),
                pl.BlockSpec((1, F1), lambda i, j: (0, 0)),
                pl.BlockSpec((F1, OP), lambda i, j: (0, 0)),
                pl.BlockSpec((1, OP), lambda i, j: (0, 0)),
            ],
            out_specs=pl.BlockSpec((batch_block, OP), lambda i, j: (i, 0)),
            scratch_shapes=[pltpu.VMEM((batch_block, H), jnp.bfloat16),
                            pltpu.VMEM((batch_block, H), jnp.float32)],
        ),
        compiler_params=pltpu.CompilerParams(
            dimension_semantics=("parallel", "arbitrary"),
            vmem_limit_bytes=100 * 1024 * 1024,
        ),
    )(x, wih_s, whh_s, b_s, w1_s, b1, w2p, b2p)

    return out[:B, :O]
